# baseline (speedup 1.0000x reference)
import functools
import jax
import jax.numpy as jnp
from jax import lax
from jax.experimental import pallas as pl
from jax.experimental.pallas import tpu as pltpu
from jax.experimental.pallas import tpu_sc as plsc

VOCAB = 100000
EMBED = 64
BATCH = 1024

_info = plsc.get_sparse_core_info()
_NC = _info.num_cores
_NS = _info.num_subcores
_NW = _NC * _NS
_BPW = BATCH // _NW


def _sc_gather(table, idx):
    mesh = plsc.VectorSubcoreMesh(core_axis_name="c", subcore_axis_name="s")

    @functools.partial(
        pl.kernel,
        mesh=mesh,
        out_type=jax.ShapeDtypeStruct((BATCH, EMBED), jnp.float32),
        scratch_types=[
            pltpu.VMEM((_BPW,), jnp.int32),
            pltpu.VMEM((_BPW, EMBED), jnp.float32),
            pltpu.SemaphoreType.DMA,
        ],
        compiler_params=pltpu.CompilerParams(use_tc_tiling_on_sc=False),
    )
    def gather_kernel(table_hbm, idx_hbm, out_hbm, idx_v, rows_v, sem):
        wid = lax.axis_index("s") * _NC + lax.axis_index("c")
        base = wid * _BPW
        pltpu.sync_copy(idx_hbm.at[pl.ds(base, _BPW)], idx_v)
        pltpu.async_copy(table_hbm.at[idx_v], rows_v, sem).wait()
        pltpu.sync_copy(rows_v, out_hbm.at[pl.ds(base, _BPW)])

    return gather_kernel(table, idx)


_TILE_V = 2048
_NT = (VOCAB + _TILE_V - 1) // _TILE_V
_LAST_W = VOCAB - (_NT - 1) * _TILE_V
_NBUF = 4


def _tc_project(emb, W, b2d):
    def mm_kernel(emb_ref, w_ref, b_ref, out_ref, obuf, tailbuf, sems, tail_sem):
        j = pl.program_id(0)
        slot = lax.rem(j, _NBUF)

        @pl.when(j >= _NBUF)
        def _():
            pltpu.make_async_copy(
                obuf.at[slot],
                out_ref.at[:, pl.ds((j - _NBUF) * _TILE_V, _TILE_V)],
                sems.at[slot],
            ).wait()

        val = jnp.broadcast_to(b_ref[...], (BATCH, _TILE_V))

        @pl.when(j < _NT - 1)
        def _():
            obuf[slot] = val
            pltpu.make_async_copy(
                obuf.at[slot],
                out_ref.at[:, pl.ds(j * _TILE_V, _TILE_V)],
                sems.at[slot],
            ).start()

        @pl.when(j == _NT - 1)
        def _():
            tailbuf[...] = val[:, :_LAST_W]
            pltpu.make_async_copy(
                tailbuf,
                out_ref.at[:, pl.ds((_NT - 1) * _TILE_V, _LAST_W)],
                tail_sem,
            ).start()
            for t in range(_NT - _NBUF, _NT - 1):
                s = t % _NBUF
                pltpu.make_async_copy(
                    obuf.at[s],
                    out_ref.at[:, pl.ds(t * _TILE_V, _TILE_V)],
                    sems.at[s],
                ).wait()
            pltpu.make_async_copy(
                tailbuf,
                out_ref.at[:, pl.ds((_NT - 1) * _TILE_V, _LAST_W)],
                tail_sem,
            ).wait()

    return pl.pallas_call(
        mm_kernel,
        grid=(_NT,),
        in_specs=[
            pl.BlockSpec((BATCH, EMBED), lambda j: (0, 0)),
            pl.BlockSpec((EMBED, _TILE_V), lambda j: (0, j)),
            pl.BlockSpec((1, _TILE_V), lambda j: (0, j)),
        ],
        out_specs=pl.BlockSpec(memory_space=pl.ANY),
        out_shape=jax.ShapeDtypeStruct((BATCH, VOCAB), jnp.float32),
        scratch_shapes=[
            pltpu.VMEM((_NBUF, BATCH, _TILE_V), jnp.float32),
            pltpu.VMEM((BATCH, _LAST_W), jnp.float32),
            pltpu.SemaphoreType.DMA((_NBUF,)),
            pltpu.SemaphoreType.DMA,
        ],
    )(emb, W, b2d)


def kernel(input, table, W, b):
    idx = input.astype(jnp.int32)
    emb = _sc_gather(table, idx)
    return _tc_project(emb, W, b.reshape(1, VOCAB))


# speedup vs baseline: 1.0013x; 1.0013x over previous
import functools
import jax
import jax.numpy as jnp
from jax import lax
from jax.experimental import pallas as pl
from jax.experimental.pallas import tpu as pltpu
from jax.experimental.pallas import tpu_sc as plsc

VOCAB = 100000
EMBED = 64
BATCH = 1024

_info = plsc.get_sparse_core_info()
_NC = _info.num_cores
_NS = _info.num_subcores
_NW = _NC * _NS
_BPW = BATCH // _NW


def _sc_gather(table, idx):
    mesh = plsc.VectorSubcoreMesh(core_axis_name="c", subcore_axis_name="s")

    @functools.partial(
        pl.kernel,
        mesh=mesh,
        out_type=jax.ShapeDtypeStruct((BATCH, EMBED), jnp.float32),
        scratch_types=[
            pltpu.VMEM((_BPW,), jnp.int32),
            pltpu.VMEM((_BPW, EMBED), jnp.float32),
            pltpu.SemaphoreType.DMA,
        ],
        compiler_params=pltpu.CompilerParams(use_tc_tiling_on_sc=False),
    )
    def gather_kernel(table_hbm, idx_hbm, out_hbm, idx_v, rows_v, sem):
        wid = lax.axis_index("s") * _NC + lax.axis_index("c")
        base = wid * _BPW
        pltpu.sync_copy(idx_hbm.at[pl.ds(base, _BPW)], idx_v)
        pltpu.async_copy(table_hbm.at[idx_v], rows_v, sem).wait()
        pltpu.sync_copy(rows_v, out_hbm.at[pl.ds(base, _BPW)])

    return gather_kernel(table, idx)


_TILE_V = 4096
_NT = (VOCAB + _TILE_V - 1) // _TILE_V
_LAST_W = VOCAB - (_NT - 1) * _TILE_V
_NBUF = 2


def _tc_project(emb, W, b2d):
    def mm_kernel(emb_ref, w_ref, b_ref, out_ref, obuf, tailbuf, sems, tail_sem):
        j = pl.program_id(0)
        slot = lax.rem(j, _NBUF)

        @pl.when(j >= _NBUF)
        def _():
            pltpu.make_async_copy(
                obuf.at[slot],
                out_ref.at[:, pl.ds((j - _NBUF) * _TILE_V, _TILE_V)],
                sems.at[slot],
            ).wait()

        val = (
            jnp.dot(emb_ref[...], w_ref[...], preferred_element_type=jnp.float32)
            + b_ref[...]
        )

        @pl.when(j < _NT - 1)
        def _():
            obuf[slot] = val
            pltpu.make_async_copy(
                obuf.at[slot],
                out_ref.at[:, pl.ds(j * _TILE_V, _TILE_V)],
                sems.at[slot],
            ).start()

        @pl.when(j == _NT - 1)
        def _():
            tailbuf[...] = val[:, :_LAST_W]
            pltpu.make_async_copy(
                tailbuf,
                out_ref.at[:, pl.ds((_NT - 1) * _TILE_V, _LAST_W)],
                tail_sem,
            ).start()
            for t in range(_NT - _NBUF, _NT - 1):
                s = t % _NBUF
                pltpu.make_async_copy(
                    obuf.at[s],
                    out_ref.at[:, pl.ds(t * _TILE_V, _TILE_V)],
                    sems.at[s],
                ).wait()
            pltpu.make_async_copy(
                tailbuf,
                out_ref.at[:, pl.ds((_NT - 1) * _TILE_V, _LAST_W)],
                tail_sem,
            ).wait()

    return pl.pallas_call(
        mm_kernel,
        grid=(_NT,),
        in_specs=[
            pl.BlockSpec((BATCH, EMBED), lambda j: (0, 0)),
            pl.BlockSpec((EMBED, _TILE_V), lambda j: (0, j)),
            pl.BlockSpec((1, _TILE_V), lambda j: (0, j)),
        ],
        out_specs=pl.BlockSpec(memory_space=pl.ANY),
        out_shape=jax.ShapeDtypeStruct((BATCH, VOCAB), jnp.float32),
        scratch_shapes=[
            pltpu.VMEM((_NBUF, BATCH, _TILE_V), jnp.float32),
            pltpu.VMEM((BATCH, _LAST_W), jnp.float32),
            pltpu.SemaphoreType.DMA((_NBUF,)),
            pltpu.SemaphoreType.DMA,
        ],
    )(emb, W, b2d)


def kernel(input, table, W, b):
    idx = input.astype(jnp.int32)
    emb = _sc_gather(table, idx)
    return _tc_project(emb, W, b.reshape(1, VOCAB))


# speedup vs baseline: 2.0590x; 2.0564x over previous
import functools
import jax
import jax.numpy as jnp
from jax import lax
from jax.experimental import pallas as pl
from jax.experimental.pallas import tpu as pltpu
from jax.experimental.pallas import tpu_sc as plsc

VOCAB = 100000
EMBED = 64
BATCH = 1024

_info = plsc.get_sparse_core_info()
_NC = _info.num_cores
_NS = _info.num_subcores
_NW = _NC * _NS
_BPW = BATCH // _NW


def _sc_gather(table, idx):
    mesh = plsc.VectorSubcoreMesh(core_axis_name="c", subcore_axis_name="s")

    @functools.partial(
        pl.kernel,
        mesh=mesh,
        out_type=jax.ShapeDtypeStruct((BATCH, EMBED), jnp.float32),
        scratch_types=[
            pltpu.VMEM((_BPW,), jnp.int32),
            pltpu.VMEM((_BPW, EMBED), jnp.float32),
            pltpu.SemaphoreType.DMA,
        ],
        compiler_params=pltpu.CompilerParams(use_tc_tiling_on_sc=False),
    )
    def gather_kernel(table_hbm, idx_hbm, out_hbm, idx_v, rows_v, sem):
        wid = lax.axis_index("s") * _NC + lax.axis_index("c")
        base = wid * _BPW
        pltpu.sync_copy(idx_hbm.at[pl.ds(base, _BPW)], idx_v)
        pltpu.async_copy(table_hbm.at[idx_v], rows_v, sem).wait()
        pltpu.sync_copy(rows_v, out_hbm.at[pl.ds(base, _BPW)])

    return gather_kernel(table, idx)


_TILE_V = 2048
_NT = (VOCAB + _TILE_V - 1) // _TILE_V


def _tc_project_t(emb, W, bcol):
    def mm_kernel(emb_ref, w_ref, b_ref, out_ref):
        out_ref[...] = (
            lax.dot_general(
                w_ref[...],
                emb_ref[...],
                (((0,), (1,)), ((), ())),
                preferred_element_type=jnp.float32,
            )
            + b_ref[...]
        )

    return pl.pallas_call(
        mm_kernel,
        grid=(_NT,),
        in_specs=[
            pl.BlockSpec((BATCH, EMBED), lambda j: (0, 0)),
            pl.BlockSpec((EMBED, _TILE_V), lambda j: (0, j)),
            pl.BlockSpec((_TILE_V, 1), lambda j: (j, 0)),
        ],
        out_specs=pl.BlockSpec((_TILE_V, BATCH), lambda j: (j, 0)),
        out_shape=jax.ShapeDtypeStruct((VOCAB, BATCH), jnp.float32),
    )(emb, W, bcol)


def kernel(input, table, W, b):
    idx = input.astype(jnp.int32)
    emb = _sc_gather(table, idx)
    logits_t = _tc_project_t(emb, W, b.reshape(VOCAB, 1))
    return logits_t.T


# speedup vs baseline: 2.6184x; 1.2717x over previous
import functools
import jax
import jax.numpy as jnp
from jax import lax
from jax.experimental import pallas as pl
from jax.experimental.pallas import tpu as pltpu
from jax.experimental.pallas import tpu_sc as plsc

VOCAB = 100000
EMBED = 64
BATCH = 1024

_info = plsc.get_sparse_core_info()
_NC = _info.num_cores
_NS = _info.num_subcores
_NW = _NC * _NS
_BPW = BATCH // _NW


def _sc_gather(table, idx):
    mesh = plsc.VectorSubcoreMesh(core_axis_name="c", subcore_axis_name="s")

    @functools.partial(
        pl.kernel,
        mesh=mesh,
        out_type=jax.ShapeDtypeStruct((BATCH, EMBED), jnp.float32),
        scratch_types=[
            pltpu.VMEM((_BPW,), jnp.int32),
            pltpu.VMEM((_BPW, EMBED), jnp.float32),
            pltpu.SemaphoreType.DMA,
        ],
        compiler_params=pltpu.CompilerParams(use_tc_tiling_on_sc=False),
    )
    def gather_kernel(table_hbm, idx_hbm, out_hbm, idx_v, rows_v, sem):
        wid = lax.axis_index("s") * _NC + lax.axis_index("c")
        base = wid * _BPW
        pltpu.sync_copy(idx_hbm.at[pl.ds(base, _BPW)], idx_v)
        pltpu.async_copy(table_hbm.at[idx_v], rows_v, sem).wait()
        pltpu.sync_copy(rows_v, out_hbm.at[pl.ds(base, _BPW)])

    return gather_kernel(table, idx)


_TILE_V = 2048
_NT = (VOCAB + _TILE_V - 1) // _TILE_V


_KAUG = EMBED + 8


def _tc_project_t(emb_aug, W, b2d):
    def mm_kernel(emb_ref, w_ref, b_ref, out_ref, waug):
        waug[0:EMBED, :] = w_ref[...]
        waug[EMBED:_KAUG, :] = jnp.concatenate(
            [b_ref[...], jnp.zeros((_KAUG - EMBED - 1, _TILE_V), jnp.float32)],
            axis=0,
        )
        out_ref[...] = lax.dot_general(
            waug[...],
            emb_ref[...],
            (((0,), (1,)), ((), ())),
            preferred_element_type=jnp.float32,
        )

    return pl.pallas_call(
        mm_kernel,
        grid=(_NT,),
        in_specs=[
            pl.BlockSpec((BATCH, _KAUG), lambda j: (0, 0)),
            pl.BlockSpec((EMBED, _TILE_V), lambda j: (0, j)),
            pl.BlockSpec((1, _TILE_V), lambda j: (0, j)),
        ],
        out_specs=pl.BlockSpec((_TILE_V, BATCH), lambda j: (j, 0)),
        out_shape=jax.ShapeDtypeStruct((VOCAB, BATCH), jnp.float32),
        scratch_shapes=[pltpu.VMEM((_KAUG, _TILE_V), jnp.float32)],
    )(emb_aug, W, b2d)


def kernel(input, table, W, b):
    idx = input.astype(jnp.int32)
    emb = _sc_gather(table, idx)
    emb_aug = jnp.concatenate(
        [emb, jnp.ones((BATCH, 1), jnp.float32),
         jnp.zeros((BATCH, _KAUG - EMBED - 1), jnp.float32)],
        axis=1,
    )
    logits_t = _tc_project_t(emb_aug, W, b.reshape(1, VOCAB))
    return logits_t.T


# speedup vs baseline: 3.3547x; 1.2812x over previous
import functools
import jax
import jax.numpy as jnp
from jax import lax
from jax.experimental import pallas as pl
from jax.experimental.pallas import tpu as pltpu
from jax.experimental.pallas import tpu_sc as plsc

VOCAB = 100000
EMBED = 64
BATCH = 1024

_info = plsc.get_sparse_core_info()
_NC = _info.num_cores
_NS = _info.num_subcores
_NW = _NC * _NS
_BPW = BATCH // _NW

_KAUG = EMBED + 8


_NBUFG = 8


def _sc_gather(table_t, idx):
    mesh = plsc.VectorSubcoreMesh(core_axis_name="c", subcore_axis_name="s")

    @functools.partial(
        pl.kernel,
        mesh=mesh,
        out_type=jax.ShapeDtypeStruct((BATCH, EMBED), jnp.float32),
        scratch_types=[
            pltpu.VMEM((_BPW,), jnp.int32),
            pltpu.VMEM((_NBUFG, EMBED, 128), jnp.float32),
            pltpu.VMEM((_BPW, EMBED), jnp.float32),
            pltpu.SemaphoreType.DMA((_NBUFG,)),
            pltpu.SemaphoreType.DMA,
        ],
        compiler_params=pltpu.CompilerParams(use_tc_tiling_on_sc=True, needs_layout_passes=False),
    )
    def gather_kernel(table_hbm, idx_hbm, out_hbm, idx_v, tbuf, rows_v, sems, osem):
        wid = lax.axis_index("s") * _NC + lax.axis_index("c")
        base = wid * _BPW
        pltpu.sync_copy(idx_hbm.at[pl.ds(base, _BPW)], idx_v)

        def tile_of(i):
            chunk = idx_v[pl.ds((i // 16) * 16, 16)]
            v = jax.lax.squeeze(jax.lax.slice(chunk, (i % 16,), (i % 16 + 1,)), (0,))
            return pl.multiple_of((v // 128) * 128, 128), v

        def dma_in(i, slot):
            col0, _ = tile_of(i)
            return pltpu.make_async_copy(
                table_hbm.at[:, pl.ds(col0, 128)], tbuf.at[slot], sems.at[slot]
            )

        for i in range(_NBUFG):
            dma_in(i, i).start()
        for i in range(_BPW):
            slot = i % _NBUFG
            dma_in(i, slot).wait()
            _, v = tile_of(i)
            lane = jax.lax.broadcast_in_dim(v % 128, (16,), ())
            for g in range(EMBED // 16):
                row = jax.lax.broadcasted_iota(jnp.int32, (16,), 0) + (g * 16)
                vals = plsc.load_gather(tbuf.at[slot], [row, lane])
                rows_v[i, pl.ds(g * 16, 16)] = vals
            if i + _NBUFG < _BPW:
                dma_in(i + _NBUFG, slot).start()
        pltpu.sync_copy(rows_v, out_hbm.at[pl.ds(base, _BPW)])

    return gather_kernel(table_t, idx)


_TILE_V = 2048
_NT = (VOCAB + _TILE_V - 1) // _TILE_V


def _tc_project_t(emb_aug, W, b2d):
    def mm_kernel(emb_ref, w_ref, b_ref, out_ref, waug):
        waug[0:EMBED, :] = w_ref[...]
        waug[EMBED:_KAUG, :] = jnp.concatenate(
            [b_ref[...], jnp.zeros((_KAUG - EMBED - 1, _TILE_V), jnp.float32)],
            axis=0,
        )
        out_ref[...] = lax.dot_general(
            waug[...],
            emb_ref[...],
            (((0,), (1,)), ((), ())),
            preferred_element_type=jnp.float32,
        )

    return pl.pallas_call(
        mm_kernel,
        grid=(_NT,),
        in_specs=[
            pl.BlockSpec((BATCH, _KAUG), lambda j: (0, 0)),
            pl.BlockSpec((EMBED, _TILE_V), lambda j: (0, j)),
            pl.BlockSpec((1, _TILE_V), lambda j: (0, j)),
        ],
        out_specs=pl.BlockSpec((_TILE_V, BATCH), lambda j: (j, 0)),
        out_shape=jax.ShapeDtypeStruct((VOCAB, BATCH), jnp.float32),
        scratch_shapes=[pltpu.VMEM((_KAUG, _TILE_V), jnp.float32)],
    )(emb_aug, W, b2d)


def kernel(input, table, W, b):
    idx = input.astype(jnp.int32)
    emb = _sc_gather(table.T, idx)
    emb_aug = jnp.concatenate(
        [emb, jnp.ones((BATCH, 1), jnp.float32),
         jnp.zeros((BATCH, _KAUG - EMBED - 1), jnp.float32)],
        axis=1,
    )
    logits_t = _tc_project_t(emb_aug, W, b.reshape(1, VOCAB))
    return logits_t.T


# speedup vs baseline: 3.4010x; 1.0138x over previous
import functools
import jax
import jax.numpy as jnp
from jax import lax
from jax.experimental import pallas as pl
from jax.experimental.pallas import tpu as pltpu
from jax.experimental.pallas import tpu_sc as plsc

VOCAB = 100000
EMBED = 64
BATCH = 1024

_info = plsc.get_sparse_core_info()
_NC = _info.num_cores
_NS = _info.num_subcores
_NW = _NC * _NS
_BPW = BATCH // _NW

_KAUG = EMBED + 8


_NBUFG = 8


def _sc_gather(table_t, idx):
    mesh = plsc.VectorSubcoreMesh(core_axis_name="c", subcore_axis_name="s")

    @functools.partial(
        pl.kernel,
        mesh=mesh,
        out_type=jax.ShapeDtypeStruct((BATCH, EMBED), jnp.float32),
        scratch_types=[
            pltpu.VMEM((_BPW,), jnp.int32),
            pltpu.VMEM((_NBUFG, EMBED, 128), jnp.float32),
            pltpu.VMEM((_BPW, EMBED), jnp.float32),
            pltpu.SemaphoreType.DMA((_NBUFG,)),
            pltpu.SemaphoreType.DMA,
        ],
        compiler_params=pltpu.CompilerParams(use_tc_tiling_on_sc=True, needs_layout_passes=False),
    )
    def gather_kernel(table_hbm, idx_hbm, out_hbm, idx_v, tbuf, rows_v, sems, osem):
        wid = lax.axis_index("s") * _NC + lax.axis_index("c")
        base = wid * _BPW
        pltpu.sync_copy(idx_hbm.at[pl.ds(base, _BPW)], idx_v)

        def tile_of(i):
            chunk = idx_v[pl.ds((i // 16) * 16, 16)]
            v = jax.lax.squeeze(jax.lax.slice(chunk, (i % 16,), (i % 16 + 1,)), (0,))
            return pl.multiple_of((v // 128) * 128, 128), v

        def dma_in(i, slot):
            col0, _ = tile_of(i)
            return pltpu.make_async_copy(
                table_hbm.at[:, pl.ds(col0, 128)], tbuf.at[slot], sems.at[slot]
            )

        for i in range(_NBUFG):
            dma_in(i, i).start()
        for i in range(_BPW):
            slot = i % _NBUFG
            dma_in(i, slot).wait()
            _, v = tile_of(i)
            lane = jax.lax.broadcast_in_dim(v % 128, (16,), ())
            for g in range(EMBED // 16):
                row = jax.lax.broadcasted_iota(jnp.int32, (16,), 0) + (g * 16)
                vals = plsc.load_gather(tbuf.at[slot], [row, lane])
                rows_v[i, pl.ds(g * 16, 16)] = vals
            if i + _NBUFG < _BPW:
                dma_in(i + _NBUFG, slot).start()
        pltpu.sync_copy(rows_v, out_hbm.at[pl.ds(base, _BPW)])

    return gather_kernel(table_t, idx)


_TILE_V = 4096
_NT = (VOCAB + _TILE_V - 1) // _TILE_V


def _tc_project_t(emb_aug, W, b2d):
    def mm_kernel(emb_ref, w_ref, b_ref, out_ref, waug):
        waug[0:EMBED, :] = w_ref[...]
        waug[EMBED:_KAUG, :] = jnp.concatenate(
            [b_ref[...], jnp.zeros((_KAUG - EMBED - 1, _TILE_V), jnp.float32)],
            axis=0,
        )
        out_ref[...] = lax.dot_general(
            waug[...],
            emb_ref[...],
            (((0,), (1,)), ((), ())),
            preferred_element_type=jnp.float32,
        )

    return pl.pallas_call(
        mm_kernel,
        grid=(_NT,),
        in_specs=[
            pl.BlockSpec((BATCH, _KAUG), lambda j: (0, 0)),
            pl.BlockSpec((EMBED, _TILE_V), lambda j: (0, j)),
            pl.BlockSpec((1, _TILE_V), lambda j: (0, j)),
        ],
        out_specs=pl.BlockSpec((_TILE_V, BATCH), lambda j: (j, 0)),
        out_shape=jax.ShapeDtypeStruct((VOCAB, BATCH), jnp.float32),
        scratch_shapes=[pltpu.VMEM((_KAUG, _TILE_V), jnp.float32)],
    )(emb_aug, W, b2d)


def kernel(input, table, W, b):
    idx = input.astype(jnp.int32)
    emb = _sc_gather(table.T, idx)
    emb_aug = jnp.concatenate(
        [emb, jnp.ones((BATCH, 1), jnp.float32),
         jnp.zeros((BATCH, _KAUG - EMBED - 1), jnp.float32)],
        axis=1,
    )
    logits_t = _tc_project_t(emb_aug, W, b.reshape(1, VOCAB))
    return logits_t.T
